# Initial kernel scaffold; baseline (speedup 1.0000x reference)
import jax
import jax.numpy as jnp
from jax.experimental import pallas as pl


def kernel(ip, table):
    raise NotImplementedError("write your pallas kernel here")


# speedup vs baseline: 1.5771x; 1.5771x over previous
import functools

import jax
import jax.numpy as jnp
from jax import lax
from jax.experimental import pallas as pl
from jax.experimental.pallas import tpu as pltpu
from jax.experimental.pallas import tpu_sc as plsc

NUM_EMB = 1000000
FEAT = 32
BATCH = 16384
FIELDS = 26
TOTAL = BATCH * FIELDS

NC = 2
NS = 16
NW = NC * NS
B_PER_W = TOTAL // NW
CHUNK = 1024
NCHUNKS = B_PER_W // CHUNK


def _embed_body(idx_hbm, table_hbm, out_hbm, idx_v, rows0, rows1, sem0, sem1):
    wid = lax.axis_index("s") * NC + lax.axis_index("c")
    base = wid * B_PER_W
    pltpu.sync_copy(idx_hbm.at[pl.ds(base, B_PER_W)], idx_v)
    bufs = (rows0, rows1)
    sems = (sem0, sem1)
    copies = [None, None]
    copies[0] = pltpu.async_copy(
        table_hbm.at[idx_v.at[pl.ds(0, CHUNK)]], bufs[0], sems[0])
    for c in range(NCHUNKS):
        cur = c % 2
        nxt = (c + 1) % 2
        if c + 1 < NCHUNKS:
            copies[nxt] = pltpu.async_copy(
                table_hbm.at[idx_v.at[pl.ds((c + 1) * CHUNK, CHUNK)]],
                bufs[nxt], sems[nxt])
        copies[cur].wait()
        pltpu.sync_copy(bufs[cur], out_hbm.at[pl.ds(base + c * CHUNK, CHUNK)])


_embed_call = pl.kernel(
    _embed_body,
    mesh=plsc.VectorSubcoreMesh(core_axis_name="c", subcore_axis_name="s"),
    out_type=jax.ShapeDtypeStruct((TOTAL, FEAT), jnp.float32),
    scratch_types=[
        pltpu.VMEM((B_PER_W,), jnp.int32),
        pltpu.VMEM((CHUNK, FEAT), jnp.float32),
        pltpu.VMEM((CHUNK, FEAT), jnp.float32),
        pltpu.SemaphoreType.DMA,
        pltpu.SemaphoreType.DMA,
    ],
    compiler_params=pltpu.CompilerParams(use_tc_tiling_on_sc=False),
)


def kernel(ip, table):
    idx = ip.reshape(TOTAL)
    out = _embed_call(idx, table)
    return out.reshape(BATCH, FIELDS, FEAT)


# speedup vs baseline: 1.5800x; 1.0019x over previous
import jax
import jax.numpy as jnp
from jax import lax
from jax.experimental import pallas as pl
from jax.experimental.pallas import tpu as pltpu
from jax.experimental.pallas import tpu_sc as plsc

NUM_EMB = 1000000
FEAT = 32
BATCH = 16384
FIELDS = 26

NC = 2
NS = 16
NW = NC * NS
B_PER_W = BATCH // NW


def _embed_body(ipt_hbm, table_hbm, out_hbm, blk_v, rows0, rows1,
                sem0, sem1):
    wid = lax.axis_index("s") * NC + lax.axis_index("c")
    b0 = wid * B_PER_W
    pltpu.sync_copy(ipt_hbm.at[:, pl.ds(b0, B_PER_W)], blk_v)
    bufs = (rows0, rows1)
    sems = (sem0, sem1)
    pltpu.async_copy(table_hbm.at[blk_v.at[0]], bufs[0], sems[0])

    def _field_pair(g, carry):
        for b in (0, 1):
            fl = 2 * g + b
            p, q = b, 1 - b

            @pl.when(fl + 1 < FIELDS)
            def _():
                pltpu.async_copy(
                    table_hbm.at[blk_v.at[fl + 1]], bufs[q], sems[q])

            pltpu.make_async_copy(
                table_hbm.at[blk_v.at[fl]], bufs[p], sems[p]).wait()
            pltpu.sync_copy(bufs[p], out_hbm.at[pl.ds(b0, B_PER_W), fl, :])
        return carry

    lax.fori_loop(0, FIELDS // 2, _field_pair, 0)


_embed_call = pl.kernel(
    _embed_body,
    mesh=plsc.VectorSubcoreMesh(core_axis_name="c", subcore_axis_name="s"),
    out_type=jax.ShapeDtypeStruct((BATCH, FIELDS, FEAT), jnp.float32),
    scratch_types=[
        pltpu.VMEM((FIELDS, B_PER_W), jnp.int32),
        pltpu.VMEM((B_PER_W, FEAT), jnp.float32),
        pltpu.VMEM((B_PER_W, FEAT), jnp.float32),
        pltpu.SemaphoreType.DMA,
        pltpu.SemaphoreType.DMA,
    ],
    compiler_params=pltpu.CompilerParams(use_tc_tiling_on_sc=False),
)


def kernel(ip, table):
    return _embed_call(ip.T, table)
